# Initial kernel scaffold; baseline (speedup 1.0000x reference)
import jax
import jax.numpy as jnp
from jax.experimental import pallas as pl


def kernel(x, edge_index, edge_attr, batch, W0, b0, Wc0, bc0, Wc1, bc1, Wc2, bc2, g0, bt0, g1, bt1, g2, bt2, W1, b1, W2, b2, W3, b3):
    raise NotImplementedError("write your pallas kernel here")


# speedup vs baseline: 8.4424x; 8.4424x over previous
import functools
import jax
import jax.numpy as jnp
from jax import lax
from jax.experimental import pallas as pl
from jax.experimental.pallas import tpu as pltpu, tpu_sc as plsc

N = 10000
E = 320000
H = 128
G = 64
EPS = 1e-5

NP = 10240
NSUB = 16
NW = 2 * NSUB
CH = 128
EPT = 10112
EPAD = EPT * NW
NCH = EPT // CH
ROWS_PT = NP // NSUB


def _sc_mesh():
    return plsc.VectorSubcoreMesh(core_axis_name="c", subcore_axis_name="s",
                                  num_cores=2, num_subcores=NSUB)


def _sc_deg_body(dst_hbm, out_hbm, idx_v, deg_v):
    cid = lax.axis_index("c")
    sid = lax.axis_index("s")
    wid = cid * NSUB + sid

    def zb(i, c):
        deg_v[pl.ds(i * 16, 16)] = jnp.zeros((16,), jnp.float32)
        return c
    lax.fori_loop(0, NP // 16, zb, 0)

    pltpu.sync_copy(dst_hbm.at[pl.ds(wid * EPT, EPT)], idx_v)
    ones = jnp.ones((16,), jnp.float32)

    def eb(j, c):
        idx = idx_v[pl.ds(j * 16, 16)]
        plsc.addupdate_scatter(deg_v, [idx], ones)
        return c
    lax.fori_loop(0, EPT // 16, eb, 0)

    pltpu.sync_copy(deg_v, out_hbm.at[pl.ds(wid * NP, NP)])


@functools.cache
def _sc_deg_kernel():
    return pl.kernel(
        _sc_deg_body,
        out_type=jax.ShapeDtypeStruct((NW * NP,), jnp.float32),
        mesh=_sc_mesh(),
        scratch_types=[
            pltpu.VMEM((EPT,), jnp.int32),
            pltpu.VMEM((NP,), jnp.float32),
        ],
        compiler_params=pltpu.CompilerParams(needs_layout_passes=False),
    )


def _sc_deg(dstp):
    return _sc_deg_kernel()(dstp)


def _sc_scat_body(y_hbm, src_hbm, dst_hbm, out_hbm,
                  sidx_v, didx_v, schunk_v, dchunk_v, rows_v, acc_sh, sem):
    cid = lax.axis_index("c")
    sid = lax.axis_index("s")
    wid = cid * NSUB + sid
    base = wid * EPT

    pltpu.sync_copy(src_hbm.at[pl.ds(base, EPT)], sidx_v)
    pltpu.sync_copy(dst_hbm.at[pl.ds(base, EPT)], didx_v)

    def zb(i, c):
        r = i // (H // 16)
        k = i % (H // 16)
        rows_v[r, pl.ds(k * 16, 16)] = jnp.zeros((16,), jnp.float32)
        return c
    lax.fori_loop(0, CH * (H // 16), zb, 0)

    def za(k, c):
        pltpu.sync_copy(rows_v, acc_sh.at[pl.ds(sid * ROWS_PT + k * CH, CH)])
        return c
    lax.fori_loop(0, ROWS_PT // CH, za, 0)
    plsc.subcore_barrier()

    def eb(j, c):
        def cp(k, c2):
            schunk_v[pl.ds(k * 16, 16)] = sidx_v[pl.ds(j * CH + k * 16, 16)]
            dchunk_v[pl.ds(k * 16, 16)] = didx_v[pl.ds(j * CH + k * 16, 16)]
            return c2
        lax.fori_loop(0, CH // 16, cp, 0)
        pltpu.async_copy(y_hbm.at[schunk_v], rows_v, sem).wait()
        pltpu.sync_copy(rows_v, acc_sh.at[dchunk_v], add=True)
        return c
    lax.fori_loop(0, NCH, eb, 0)
    plsc.subcore_barrier()

    pltpu.sync_copy(acc_sh.at[pl.ds(sid * ROWS_PT, ROWS_PT)],
                    out_hbm.at[pl.ds(cid * NP + sid * ROWS_PT, ROWS_PT)])


@functools.cache
def _sc_scat_kernel():
    return pl.kernel(
        _sc_scat_body,
        out_type=jax.ShapeDtypeStruct((2 * NP, H), jnp.float32),
        mesh=_sc_mesh(),
        scratch_types=[
            pltpu.VMEM((EPT,), jnp.int32),
            pltpu.VMEM((EPT,), jnp.int32),
            pltpu.VMEM((CH,), jnp.int32),
            pltpu.VMEM((CH,), jnp.int32),
            pltpu.VMEM((CH, H), jnp.float32),
            pltpu.VMEM_SHARED((NP, H), jnp.float32),
            pltpu.SemaphoreType.DMA,
        ],
    )


def _sc_scat(y, srcp, dstp):
    return _sc_scat_kernel()(y, srcp, dstp)


def _lrelu(v):
    return jnp.where(v >= 0, v, 0.01 * v)


def _onehots(bcol, brow):
    iota_row = lax.broadcasted_iota(jnp.int32, (1, G), 1)
    iota_col = lax.broadcasted_iota(jnp.int32, (G, 1), 0)
    oh = (bcol == iota_row).astype(jnp.float32)
    oh_t = (brow == iota_col).astype(jnp.float32)
    return oh, oh_t


def _dot(a, b):
    return jnp.dot(a, b, preferred_element_type=jnp.float32)


def _write_y(y_ref, y):
    y_ref[0:N, :] = y
    y_ref[N:NP, :] = jnp.zeros((NP - N, H), jnp.float32)


def _gmp(h, bcol, cnt):
    iota_col = lax.broadcasted_iota(jnp.int32, (G, 1), 0)
    neg = jnp.float32(-3.4e38)

    def body(g, acc):
        m = jnp.max(jnp.where(bcol == g, h, neg), axis=0, keepdims=True)
        return jnp.where(iota_col == g, m, acc)
    m = lax.fori_loop(0, G, body, jnp.zeros((G, H), jnp.float32))
    return jnp.where(cnt > 0, m, 0.0)


def _tc_pre_body(x_r, bcol_r, brow_r, degp_r, w0_r, b0_r, wt_r, wb_r,
                 y_r, dis_r):
    deg = jnp.sum(degp_r[...], axis=1, keepdims=True) + 1.0
    dis = lax.rsqrt(deg)
    dis_r[...] = dis

    h = _lrelu(_dot(x_r[...], w0_r[...]) + b0_r[...])
    oh, oh_t = _onehots(bcol_r[...], brow_r[...])
    cnt = jnp.sum(oh_t, axis=1, keepdims=True)
    gap = _dot(oh_t, h) / jnp.maximum(cnt, 1.0)
    xw = _dot(h, wt_r[...]) + _dot(oh, _dot(gap, wb_r[...]))
    _write_y(y_r, dis[0:N] * xw)


def _tc_pre(x, bcol, brow, degp, w0, b0, wt, wb):
    return pl.pallas_call(
        _tc_pre_body,
        out_shape=[
            jax.ShapeDtypeStruct((NP, H), jnp.float32),
            jax.ShapeDtypeStruct((NP, 1), jnp.float32),
        ],
    )(x, bcol, brow, degp, w0, b0, wt, wb)


def _layer_tail(z0, z1, y, dis, bcol, brow, bc, gamma, beta):
    z = z0 + z1 + y
    gcn = dis[0:N] * z[0:N] + bc
    oh, oh_t = _onehots(bcol, brow)
    cnt = jnp.sum(oh_t, axis=1, keepdims=True)
    denom = jnp.maximum(cnt, 1.0) * H
    sg = _dot(oh_t, gcn)
    mean = jnp.sum(sg, axis=1, keepdims=True) / denom
    q = jnp.sum(gcn * gcn, axis=1, keepdims=True)
    s2 = _dot(oh_t, q)
    var = s2 / denom - mean * mean
    mean_n = _dot(oh, mean)
    inv_n = _dot(oh, lax.rsqrt(var + EPS))
    xn = (gcn - mean_n) * inv_n * gamma + beta
    h = _lrelu(xn)
    gapv = _dot(oh_t, h) / jnp.maximum(cnt, 1.0)
    gmpv = _gmp(h, bcol, cnt)
    return h, oh, cnt, gapv, gmpv


def _tc_mid_body(z0_r, z1_r, y_r, dis_r, bcol_r, brow_r, pooled_r,
                 bc_r, g_r, bt_r, wt_r, wb_r, ynext_r, pout_r):
    h, oh, cnt, gapv, gmpv = _layer_tail(
        z0_r[...], z1_r[...], y_r[...], dis_r[...], bcol_r[...], brow_r[...],
        bc_r[...], g_r[...], bt_r[...])
    pout_r[...] = pooled_r[...] + jnp.concatenate([gmpv, gapv], axis=1)
    xw = _dot(h, wt_r[...]) + _dot(oh, _dot(gapv, wb_r[...]))
    _write_y(ynext_r, dis_r[0:N] * xw)


def _tc_mid(z0, z1, y, dis, bcol, brow, pooled, bc, g, bt, wt, wb):
    return pl.pallas_call(
        _tc_mid_body,
        out_shape=[
            jax.ShapeDtypeStruct((NP, H), jnp.float32),
            jax.ShapeDtypeStruct((G, 2 * H), jnp.float32),
        ],
    )(z0, z1, y, dis, bcol, brow, pooled, bc, g, bt, wt, wb)


def _tc_final_body(z0_r, z1_r, y_r, dis_r, bcol_r, brow_r, pooled_r,
                   bc_r, g_r, bt_r, w1_r, b1_r, w2_r, b2_r, w3_r, b3_r,
                   out_r):
    h, oh, cnt, gapv, gmpv = _layer_tail(
        z0_r[...], z1_r[...], y_r[...], dis_r[...], bcol_r[...], brow_r[...],
        bc_r[...], g_r[...], bt_r[...])
    pooled = pooled_r[...] + jnp.concatenate([gmpv, gapv], axis=1)
    o = _lrelu(_dot(pooled, w1_r[...]) + b1_r[...])
    o = _lrelu(_dot(o, w2_r[...]) + b2_r[...])
    out_r[...] = _dot(o, w3_r[...]) + b3_r[...]


def _tc_final(z0, z1, y, dis, bcol, brow, pooled, bc, g, bt,
              w1, b1, w2, b2, w3, b3):
    return pl.pallas_call(
        _tc_final_body,
        out_shape=jax.ShapeDtypeStruct((G, 1), jnp.float32),
    )(z0, z1, y, dis, bcol, brow, pooled, bc, g, bt, w1, b1, w2, b2, w3, b3)


@jax.jit
def kernel(x, edge_index, edge_attr, batch,
           W0, b0, Wc0, bc0, Wc1, bc1, Wc2, bc2,
           g0, bt0, g1, bt1, g2, bt2, W1, b1, W2, b2, W3, b3):
    del edge_attr
    src = edge_index[0]
    dst = edge_index[1]
    pad = jnp.full((EPAD - E,), N, jnp.int32)
    srcp = jnp.concatenate([src, pad])
    dstp = jnp.concatenate([dst, pad])

    bcol = batch.reshape(N, 1)
    brow = batch.reshape(1, N)

    degp = _sc_deg(dstp)
    degp_t = degp.reshape(NW, NP).T

    y0, dis = _tc_pre(x, bcol, brow, degp_t, W0, b0.reshape(1, H),
                      Wc0[:H], Wc0[H:])

    pooled = jnp.zeros((G, 2 * H), jnp.float32)
    y = y0
    per_layer = [
        (bc0, g0, bt0, Wc1),
        (bc1, g1, bt1, Wc2),
    ]
    for bc, g, bt, wnext in per_layer:
        zf = _sc_scat(y, srcp, dstp).reshape(2, NP, H)
        y, pooled = _tc_mid(zf[0], zf[1], y, dis, bcol, brow, pooled,
                            bc.reshape(1, H), g.reshape(1, H),
                            bt.reshape(1, H), wnext[:H], wnext[H:])

    zf = _sc_scat(y, srcp, dstp).reshape(2, NP, H)
    out = _tc_final(zf[0], zf[1], y, dis, bcol, brow, pooled,
                    bc2.reshape(1, H), g2.reshape(1, H), bt2.reshape(1, H),
                    W1, b1.reshape(1, 4 * H), W2, b2.reshape(1, 4 * H),
                    W3, b3.reshape(1, 1))
    return out
